# baseline (speedup 1.0000x reference)
import functools

import jax
import jax.numpy as jnp
from jax import lax
from jax.experimental import pallas as pl
from jax.experimental.pallas import tpu as pltpu
from jax.experimental.pallas import tpu_sc as plsc


def _build(B, L, SL, N):
    NC, NS = 2, 16
    NW = NC * NS
    assert B % NW == 0
    rows_per_worker = B // NW
    RB = 16
    assert rows_per_worker % RB == 0
    chunks = rows_per_worker // RB
    WB = L // 32
    WSH = (L // 32).bit_length() - 1
    JV = SL // 16
    NBUF = 8
    IQ = N // 4

    mesh = plsc.VectorSubcoreMesh(core_axis_name="c", subcore_axis_name="s")

    @functools.partial(
        pl.kernel,
        out_type=jax.ShapeDtypeStruct((N, B, SL), jnp.float32),
        mesh=mesh,
        scratch_types=[pltpu.VMEM((RB, L // 32), jnp.int32)] * 2
          + [pltpu.VMEM((RB, SL), jnp.float32)] * 2
          + [pltpu.VMEM((N,), jnp.int32)]
          + [pltpu.VMEM((RB, SL), jnp.float32)] * NBUF
          + [pltpu.SemaphoreType.DMA] * (NBUF + 2),
        compiler_params=pltpu.CompilerParams(needs_layout_passes=False),
    )
    def k(w_hbm, s_hbm, idx_hbm, out_hbm, wrows0, wrows1, srows0, srows1,
          svmem, *obs_sems):
        obs = obs_sems[:NBUF]
        sems = obs_sems[NBUF:NBUF * 2]
        isems = obs_sems[NBUF * 2:]
        wid = lax.axis_index("s") * NC + lax.axis_index("c")
        pltpu.sync_copy(idx_hbm, svmem)
        lane = lax.iota(jnp.int32, 16)
        zeros16 = jnp.zeros((16,), jnp.int32)
        base0 = wid * rows_per_worker

        def compute_quad(i0, obg, wrows, srows):
            s_vecs = [plsc.load_gather(svmem, [zeros16 + (i0 + q)])
                      for q in range(4)]

            @plsc.parallel_loop(0, JV)
            def _(j):
                t16 = j * 16 + lane
                pws = []
                for q in range(4):
                    pos = s_vecs[q] + t16
                    pws.append((pos & (WB - 1),
                                jnp.int32(1) << (pos >> WSH)))
                for bl in range(RB):
                    vals = srows[bl, pl.ds(j * 16, 16)]
                    for q in range(4):
                        w = plsc.load_gather(
                            wrows, [zeros16 + bl, pws[q][0]])
                        obg[q][bl, pl.ds(j * 16, 16)] = jnp.where(
                            (w & pws[q][1]) != 0, vals, 0.0)

        pltpu.async_copy(w_hbm.at[pl.ds(base0, RB)], wrows0, isems[0])
        pltpu.async_copy(s_hbm.at[pl.ds(base0, RB)], srows0, isems[0])
        pltpu.async_copy(w_hbm.at[pl.ds(base0 + RB, RB)], wrows1, isems[1])
        pltpu.async_copy(s_hbm.at[pl.ds(base0 + RB, RB)], srows1, isems[1])

        def cp_body(cp, _):
            for half, (wr, sr, isem) in enumerate(
                    ((wrows0, srows0, isems[0]), (wrows1, srows1, isems[1]))):
                c = 2 * cp + half
                base = base0 + c * RB
                pltpu.make_async_copy(
                    w_hbm.at[pl.ds(0, RB)], wr, isem).wait()
                pltpu.make_async_copy(
                    s_hbm.at[pl.ds(0, RB)], sr, isem).wait()

                def iq_body(iq2, _, c=c, base=base, wr=wr, sr=sr,
                            half=half):
                    for g in range(2):
                        iq = 2 * iq2 + g
                        grp = obs[4 * g:4 * g + 4]
                        gsems = sems[4 * g:4 * g + 4]
                        for q in range(4):
                            if half == 0:
                                @pl.when((cp > 0) | (iq2 > 0))
                                def _(q=q, grp=grp, gsems=gsems):
                                    pltpu.make_async_copy(
                                        grp[q], out_hbm.at[0, pl.ds(0, RB)],
                                        gsems[q]).wait()
                            else:
                                pltpu.make_async_copy(
                                    grp[q], out_hbm.at[0, pl.ds(0, RB)],
                                    gsems[q]).wait()
                        compute_quad(4 * iq, grp, wr, sr)
                        for q in range(4):
                            pltpu.async_copy(
                                grp[q],
                                out_hbm.at[4 * iq + q, pl.ds(base, RB)],
                                gsems[q])
                    return 0

                lax.fori_loop(0, IQ // 2, iq_body, 0)

                @pl.when(c + 2 < chunks)
                def _(base=base, wr=wr, sr=sr, isem=isem):
                    pltpu.async_copy(
                        w_hbm.at[pl.ds(base + 2 * RB, RB)], wr, isem)
                    pltpu.async_copy(
                        s_hbm.at[pl.ds(base + 2 * RB, RB)], sr, isem)
            return 0

        lax.fori_loop(0, chunks // 2, cp_body, 0)
        for q in range(NBUF):
            pltpu.make_async_copy(
                obs[q], out_hbm.at[0, pl.ds(0, RB)], sems[q]).wait()

    return k


def kernel(where_input, slice_input, slice_len, start_indices):
    B, L = where_input.shape
    SL = slice_input.shape[1]
    N = start_indices.shape[0]
    zero_off = (jnp.asarray(slice_len) - SL).astype(jnp.int32)
    starts = jnp.clip(
        start_indices.astype(jnp.int32) + zero_off, 0, L - SL)
    WB = L // 32
    wbits = jnp.where(where_input[:, :WB], jnp.int32(1), jnp.int32(0))
    for kk in range(1, 32):
        wbits = wbits | jnp.where(
            where_input[:, kk * WB:(kk + 1) * WB], jnp.int32(1) << kk,
            jnp.int32(0))
    return _build(B, L, SL, N)(wbits, slice_input, starts)


# speedup vs baseline: 1.1555x; 1.1555x over previous
import functools

import jax
import jax.numpy as jnp
from jax import lax
from jax.experimental import pallas as pl
from jax.experimental.pallas import tpu as pltpu
from jax.experimental.pallas import tpu_sc as plsc


def _build(B, L, SL, N):
    NC, NS = 2, 16
    NW = NC * NS
    assert B % NW == 0
    rows_per_worker = B // NW
    RB = 16
    assert rows_per_worker % RB == 0
    chunks = rows_per_worker // RB
    WB = L // 32
    WSH = (L // 32).bit_length() - 1
    JV = SL // 16
    NBUF = 4
    IQ = N // NBUF

    mesh = plsc.VectorSubcoreMesh(core_axis_name="c", subcore_axis_name="s")

    @functools.partial(
        pl.kernel,
        out_type=jax.ShapeDtypeStruct((N, B, SL), jnp.float32),
        mesh=mesh,
        scratch_types=[pltpu.VMEM((RB, L // 32), jnp.int32)] * 2
          + [pltpu.VMEM((RB, SL), jnp.float32)] * 2
          + [pltpu.VMEM((N,), jnp.int32)]
          + [pltpu.VMEM((RB, SL), jnp.float32)] * NBUF
          + [pltpu.SemaphoreType.DMA] * (NBUF + 2),
        compiler_params=pltpu.CompilerParams(needs_layout_passes=False),
    )
    def k(w_hbm, s_hbm, idx_hbm, out_hbm, wrows0, wrows1, srows0, srows1,
          svmem, *obs_sems):
        obs = obs_sems[:NBUF]
        sems = obs_sems[NBUF:NBUF * 2]
        isems = obs_sems[NBUF * 2:]
        wid = lax.axis_index("s") * NC + lax.axis_index("c")
        pltpu.sync_copy(idx_hbm, svmem)
        lane = lax.iota(jnp.int32, 16)
        zeros16 = jnp.zeros((16,), jnp.int32)
        base0 = wid * rows_per_worker

        def compute_i(i, ob, wrows, srows):
            s_vec = plsc.load_gather(svmem, [zeros16 + i])

            @plsc.parallel_loop(0, JV // 2)
            def _(j):
                pos = s_vec + j * 16 + lane
                widx = pos & (WB - 1)
                bm0 = jnp.int32(1) << (pos >> WSH)
                bm1 = bm0 << 1
                off1 = j * 16 + (JV // 2) * 16
                for bl in range(RB):
                    w = plsc.load_gather(wrows, [zeros16 + bl, widx])
                    ob[bl, pl.ds(j * 16, 16)] = jnp.where(
                        (w & bm0) != 0, srows[bl, pl.ds(j * 16, 16)], 0.0)
                    ob[bl, pl.ds(off1, 16)] = jnp.where(
                        (w & bm1) != 0, srows[bl, pl.ds(off1, 16)], 0.0)

        pltpu.async_copy(w_hbm.at[pl.ds(base0, RB)], wrows0, isems[0])
        pltpu.async_copy(s_hbm.at[pl.ds(base0, RB)], srows0, isems[0])
        pltpu.async_copy(w_hbm.at[pl.ds(base0 + RB, RB)], wrows1, isems[1])
        pltpu.async_copy(s_hbm.at[pl.ds(base0 + RB, RB)], srows1, isems[1])

        def cp_body(cp, _):
            for half, (wr, sr, isem) in enumerate(
                    ((wrows0, srows0, isems[0]), (wrows1, srows1, isems[1]))):
                c = 2 * cp + half
                base = base0 + c * RB
                pltpu.make_async_copy(
                    w_hbm.at[pl.ds(0, RB)], wr, isem).wait()
                pltpu.make_async_copy(
                    s_hbm.at[pl.ds(0, RB)], sr, isem).wait()

                def iq_body(iq, _, c=c, base=base, wr=wr, sr=sr, half=half):
                    for q in range(NBUF):
                        if half == 0:
                            @pl.when((cp > 0) | (iq > 0))
                            def _(q=q):
                                pltpu.make_async_copy(
                                    obs[q], out_hbm.at[0, pl.ds(0, RB)],
                                    sems[q]).wait()
                        else:
                            pltpu.make_async_copy(
                                obs[q], out_hbm.at[0, pl.ds(0, RB)],
                                sems[q]).wait()
                        compute_i(NBUF * iq + q, obs[q], wr, sr)
                        pltpu.async_copy(
                            obs[q],
                            out_hbm.at[NBUF * iq + q, pl.ds(base, RB)],
                            sems[q])
                    return 0

                lax.fori_loop(0, IQ, iq_body, 0)

                @pl.when(c + 2 < chunks)
                def _(base=base, wr=wr, sr=sr, isem=isem):
                    pltpu.async_copy(
                        w_hbm.at[pl.ds(base + 2 * RB, RB)], wr, isem)
                    pltpu.async_copy(
                        s_hbm.at[pl.ds(base + 2 * RB, RB)], sr, isem)
            return 0

        lax.fori_loop(0, chunks // 2, cp_body, 0)
        for q in range(NBUF):
            pltpu.make_async_copy(
                obs[q], out_hbm.at[0, pl.ds(0, RB)], sems[q]).wait()

    return k


def kernel(where_input, slice_input, slice_len, start_indices):
    B, L = where_input.shape
    SL = slice_input.shape[1]
    N = start_indices.shape[0]
    zero_off = (jnp.asarray(slice_len) - SL).astype(jnp.int32)
    starts = jnp.clip(
        start_indices.astype(jnp.int32) + zero_off, 0, L - SL)
    WB = L // 32
    wbits = jnp.where(where_input[:, :WB], jnp.int32(1), jnp.int32(0))
    for kk in range(1, 32):
        wbits = wbits | jnp.where(
            where_input[:, kk * WB:(kk + 1) * WB], jnp.int32(1) << kk,
            jnp.int32(0))
    return _build(B, L, SL, N)(wbits, slice_input, starts)


# speedup vs baseline: 1.7916x; 1.5506x over previous
import functools

import jax
import jax.numpy as jnp
from jax import lax
from jax.experimental import pallas as pl
from jax.experimental.pallas import tpu as pltpu
from jax.experimental.pallas import tpu_sc as plsc


def _build(B, L, SL, N):
    NC, NS = 2, 16
    NW = NC * NS
    assert B % NW == 0
    rows_per_worker = B // NW
    RB = 16
    assert rows_per_worker % RB == 0
    chunks = rows_per_worker // RB
    WB = L // 32
    WSH = (L // 32).bit_length() - 1
    JV = SL // 16
    NBUF = 2
    IQ = N // NBUF

    mesh = plsc.VectorSubcoreMesh(core_axis_name="c", subcore_axis_name="s")

    @functools.partial(
        pl.kernel,
        out_type=jax.ShapeDtypeStruct((N, B, SL), jnp.float32),
        mesh=mesh,
        scratch_types=[pltpu.VMEM((RB, L // 32), jnp.int32)] * 2
          + [pltpu.VMEM((RB, SL), jnp.float32)] * 2
          + [pltpu.VMEM((N,), jnp.int32)]
          + [pltpu.VMEM((RB, SL), jnp.float32)] * NBUF
          + [pltpu.SemaphoreType.DMA] * (NBUF + 2),
        compiler_params=pltpu.CompilerParams(needs_layout_passes=False),
    )
    def k(w_hbm, s_hbm, idx_hbm, out_hbm, wrows0, wrows1, srows0, srows1,
          svmem, *obs_sems):
        obs = obs_sems[:NBUF]
        sems = obs_sems[NBUF:NBUF * 2]
        isems = obs_sems[NBUF * 2:]
        wid = lax.axis_index("s") * NC + lax.axis_index("c")
        pltpu.sync_copy(idx_hbm, svmem)
        lane = lax.iota(jnp.int32, 16)
        zeros16 = jnp.zeros((16,), jnp.int32)
        base0 = wid * rows_per_worker

        def compute_i(i, ob, wrows, srows):
            s_vec = plsc.load_gather(svmem, [zeros16 + i])

            @plsc.parallel_loop(0, JV // 2)
            def _(j):
                pos = s_vec + j * 16 + lane
                widx = pos & (WB - 1)
                bm0 = jnp.int32(1) << (pos >> WSH)
                bm1 = bm0 << 1
                off1 = j * 16 + (JV // 2) * 16
                for bl in range(RB):
                    w = plsc.load_gather(wrows, [zeros16 + bl, widx])
                    ob[bl, pl.ds(j * 16, 16)] = jnp.where(
                        (w & bm0) != 0, srows[bl, pl.ds(j * 16, 16)], 0.0)
                    ob[bl, pl.ds(off1, 16)] = jnp.where(
                        (w & bm1) != 0, srows[bl, pl.ds(off1, 16)], 0.0)

        pltpu.async_copy(w_hbm.at[pl.ds(base0, RB)], wrows0, isems[0])
        pltpu.async_copy(s_hbm.at[pl.ds(base0, RB)], srows0, isems[0])
        pltpu.async_copy(w_hbm.at[pl.ds(base0 + RB, RB)], wrows1, isems[1])
        pltpu.async_copy(s_hbm.at[pl.ds(base0 + RB, RB)], srows1, isems[1])

        def cp_body(cp, _):
            for half, (wr, sr, isem) in enumerate(
                    ((wrows0, srows0, isems[0]), (wrows1, srows1, isems[1]))):
                c = 2 * cp + half
                base = base0 + c * RB
                pltpu.make_async_copy(
                    w_hbm.at[pl.ds(0, RB)], wr, isem).wait()
                pltpu.make_async_copy(
                    s_hbm.at[pl.ds(0, RB)], sr, isem).wait()

                def iq_body(iq, _, c=c, base=base, wr=wr, sr=sr, half=half):
                    for q in range(NBUF):
                        if half == 0:
                            @pl.when((cp > 0) | (iq > 0))
                            def _(q=q):
                                pltpu.make_async_copy(
                                    obs[q], out_hbm.at[0, pl.ds(0, RB)],
                                    sems[q]).wait()
                        else:
                            pltpu.make_async_copy(
                                obs[q], out_hbm.at[0, pl.ds(0, RB)],
                                sems[q]).wait()
                        compute_i(NBUF * iq + q, obs[q], wr, sr)
                        pltpu.async_copy(
                            obs[q],
                            out_hbm.at[NBUF * iq + q, pl.ds(base, RB)],
                            sems[q])
                    return 0

                lax.fori_loop(0, IQ, iq_body, 0)

                @pl.when(c + 2 < chunks)
                def _(base=base, wr=wr, sr=sr, isem=isem):
                    pltpu.async_copy(
                        w_hbm.at[pl.ds(base + 2 * RB, RB)], wr, isem)
                    pltpu.async_copy(
                        s_hbm.at[pl.ds(base + 2 * RB, RB)], sr, isem)
            return 0

        lax.fori_loop(0, chunks // 2, cp_body, 0)
        for q in range(NBUF):
            pltpu.make_async_copy(
                obs[q], out_hbm.at[0, pl.ds(0, RB)], sems[q]).wait()

    return k


def kernel(where_input, slice_input, slice_len, start_indices):
    B, L = where_input.shape
    SL = slice_input.shape[1]
    N = start_indices.shape[0]
    zero_off = (jnp.asarray(slice_len) - SL).astype(jnp.int32)
    starts = jnp.clip(
        start_indices.astype(jnp.int32) + zero_off, 0, L - SL)
    WB = L // 32
    wbits = jnp.where(where_input[:, :WB], jnp.int32(1), jnp.int32(0))
    for kk in range(1, 32):
        wbits = wbits | jnp.where(
            where_input[:, kk * WB:(kk + 1) * WB], jnp.int32(1) << kk,
            jnp.int32(0))
    return _build(B, L, SL, N)(wbits, slice_input, starts)
